# baseline (speedup 1.0000x reference)
import functools
import math

import jax
import jax.numpy as jnp
from jax import lax
from jax.experimental import pallas as pl
from jax.experimental.pallas import tpu as pltpu
from jax.experimental.pallas import tpu_sc as plsc

EMB = 64
SCALE = math.sqrt(EMB)
LANES = 16

NC = 2
NS = 16
NW = NC * NS

G = 128
NBUF = 4


def _body(tok_hbm, table_hbm, out_hbm, idx_v, *rows_and_sems):
    rows = rows_and_sems[:NBUF]
    sems = rows_and_sems[NBUF:]
    ng = tok_hbm.shape[1]
    per_w = ng * G

    wid = lax.axis_index("s") * NC + lax.axis_index("c")
    base = wid * per_w

    pltpu.sync_copy(tok_hbm.at[wid], idx_v)

    for b in range(NBUF):
        pltpu.async_copy(table_hbm.at[idx_v.at[b]], rows[b], sems[b])

    @pl.loop(0, ng, step=NBUF)
    def _chunks(t):
        for b in range(NBUF):
            g = t + b
            pltpu.make_async_copy(
                table_hbm.at[idx_v.at[g]], rows[b], sems[b]
            ).wait()

            @plsc.parallel_loop(0, G, unroll=4)
            def _scale(r):
                for j in range(EMB // LANES):
                    sl = pl.ds(j * LANES, LANES)
                    rows[b][r, sl] = rows[b][r, sl] * SCALE

            pltpu.sync_copy(rows[b], out_hbm.at[pl.ds(base + g * G, G)])

            gn = g + NBUF

            @pl.when(gn < ng)
            def _():
                pltpu.async_copy(table_hbm.at[idx_v.at[gn]], rows[b], sems[b])


def kernel(tokens, table):
    n_tok = tokens.shape[0] * tokens.shape[1]
    per_w = n_tok // NW
    ng = per_w // G

    mesh = plsc.VectorSubcoreMesh(core_axis_name="c", subcore_axis_name="s")
    run = pl.kernel(
        _body,
        out_type=jax.ShapeDtypeStruct((n_tok, EMB), jnp.float32),
        mesh=mesh,
        compiler_params=pltpu.CompilerParams(use_tc_tiling_on_sc=False),
        scratch_types=(
            [pltpu.VMEM((ng, G), jnp.int32)]
            + [pltpu.VMEM((G, EMB), jnp.float32) for _ in range(NBUF)]
            + [pltpu.SemaphoreType.DMA for _ in range(NBUF)]
        ),
    )
    tok = tokens.reshape(NW, ng, G).astype(jnp.int32)
    out = run(tok, table)
    return out.reshape(tokens.shape[0], tokens.shape[1], EMB)
